# baseline (speedup 1.0000x reference)
import functools

import jax
import jax.numpy as jnp
from jax import lax
from jax.experimental import pallas as pl
from jax.experimental.pallas import tpu as pltpu

L = 2048
DIM = 1024
DIN = 2048
DSTATE = 16
DTRANK = 64
DCONV = 4
NEXP = 8
FFI = 2048
LC = 64
RC = 256

_F32 = jnp.float32
_NT = (((1,), (1,)), ((), ()))


def _silu(v):
    return v * jax.nn.sigmoid(v)


def _softplus(v):
    return jnp.maximum(v, 0.0) + jnp.log1p(jnp.exp(-jnp.abs(v)))


def _ntdot(a, b):
    return lax.dot_general(a, b, _NT, preferred_element_type=_F32)


def _mamba_body(x_ref, w_in_ref, conv_w_ref, conv_b_ref, wx_ref,
                w_dt_ref, b_dt_ref, alog_ref, dd_ref, w_out_ref,
                out_ref, tail_ref, state_ref, da_s, dbu_s, st_s):
    c = pl.program_id(0)

    @pl.when(c == 0)
    def _():
        tail_ref[...] = jnp.zeros_like(tail_ref)
        state_ref[...] = jnp.zeros_like(state_ref)

    xch = x_ref[...]
    xz = _ntdot(xch, w_in_ref[...])
    xc = xz[:, :DIN]
    res = xz[:, DIN:]

    ext = jnp.concatenate([tail_ref[...], xc], axis=0)
    tail_ref[...] = xc[LC - (DCONV - 1):, :]
    conv = conv_b_ref[...]
    for k in range(DCONV):
        conv = conv + ext[k:k + LC, :] * conv_w_ref[k:k + 1, :]
    xcs = _silu(conv)

    x_dbl = _ntdot(xcs, wx_ref[...])
    delta_r = x_dbl[:, :DTRANK]
    bm = x_dbl[:, DTRANK:DTRANK + DSTATE]
    cm = x_dbl[:, DTRANK + DSTATE:]
    delta = _softplus(_ntdot(delta_r, w_dt_ref[...]) + b_dt_ref[...])
    u = delta * xcs

    a2 = -jnp.exp(alog_ref[...])
    da_s[...] = jnp.exp(delta[:, None, :] * a2[None, :, :])
    dbu_s[...] = bm[:, :, None] * u[:, None, :]

    def step(l, _):
        st = (da_s[pl.ds(l, 1)][0] * state_ref[...]
              + dbu_s[pl.ds(l, 1)][0])
        state_ref[...] = st
        st_s[pl.ds(l, 1)] = st[None]
        return 0

    lax.fori_loop(0, LC, step, 0, unroll=False)

    y = jnp.sum(st_s[...] * cm[:, :, None], axis=1)
    y = y + xcs * dd_ref[...]
    y = y * _silu(res)
    out_ref[...] = _ntdot(y, w_out_ref[...]) + xch


def _mamba_block(h, bp):
    conv_w_t = bp['conv_w'].T
    conv_b = bp['conv_b'].reshape(1, DIN)
    b_dt = bp['b_dt'].reshape(1, DIN)
    alog_t = bp['A_log'].T
    dd = bp['D'].reshape(1, DIN)

    grid = L // LC
    full = lambda shape: pl.BlockSpec(shape, lambda c: (0,) * len(shape))
    return pl.pallas_call(
        _mamba_body,
        grid=(grid,),
        in_specs=[
            pl.BlockSpec((LC, DIM), lambda c: (c, 0)),
            full((2 * DIN, DIM)),
            full((DCONV, DIN)),
            full((1, DIN)),
            full((DTRANK + 2 * DSTATE, DIN)),
            full((DIN, DTRANK)),
            full((1, DIN)),
            full((DSTATE, DIN)),
            full((1, DIN)),
            full((DIM, DIN)),
        ],
        out_specs=pl.BlockSpec((LC, DIM), lambda c: (c, 0)),
        out_shape=jax.ShapeDtypeStruct((L, DIM), _F32),
        scratch_shapes=[
            pltpu.VMEM((DCONV - 1, DIN), _F32),
            pltpu.VMEM((DSTATE, DIN), _F32),
            pltpu.VMEM((LC, DSTATE, DIN), _F32),
            pltpu.VMEM((LC, DSTATE, DIN), _F32),
            pltpu.VMEM((LC, DSTATE, DIN), _F32),
        ],
    )(h, bp['W_in'], conv_w_t, conv_b, bp['W_x'], bp['W_dt'], b_dt,
      alog_t, dd, bp['W_out'])


TILE = 256
NT_MAX = L // TILE
NDATA = 2 * L + NEXP * TILE
DUMMY = NDATA // TILE
TOTAL = NDATA + TILE
NW = 32


def _router_body(h_ref, wgate_ref, e1_ref, e2_ref, rk1_ref, rk2_ref,
                 w1_ref, w2_ref, ps16_ref, pst_ref, nt_ref, base_s):
    c = pl.program_id(0)

    h = h_ref[...]
    scores = _ntdot(h, wgate_ref[...])
    ii = lax.broadcasted_iota(jnp.int32, scores.shape, 1)
    m1 = jnp.max(scores, axis=-1, keepdims=True)
    a1 = jnp.min(jnp.where(scores == m1, ii, NEXP), axis=-1, keepdims=True)
    s2 = jnp.where(ii == a1, -jnp.inf, scores)
    m2 = jnp.max(s2, axis=-1, keepdims=True)
    a2 = jnp.min(jnp.where(s2 == m2, ii, NEXP), axis=-1, keepdims=True)
    oh1 = (ii == a1).astype(_F32)
    oh2 = (ii == a2).astype(_F32)

    @pl.when(c == 0)
    def _():
        base_s[...] = jnp.zeros_like(base_s)

    rr = lax.broadcasted_iota(jnp.int32, (RC, RC), 0)
    cc = lax.broadcasted_iota(jnp.int32, (RC, RC), 1)
    tri = (cc < rr).astype(_F32)
    both = oh1 + oh2
    prev = lax.dot_general(tri, both, (((1,), (0,)), ((), ())),
                           preferred_element_type=_F32)
    r1pos = base_s[...] + prev
    r2pos = r1pos + oh1
    rk1 = jnp.sum(oh1 * r1pos, axis=1, keepdims=True)
    rk2 = jnp.sum(oh2 * r2pos, axis=1, keepdims=True)
    e1_ref[...] = a1
    e2_ref[...] = a2
    rk1_ref[...] = rk1.astype(jnp.int32)
    rk2_ref[...] = rk2.astype(jnp.int32)
    e2w = jnp.exp(m2 - m1)
    w1 = 1.0 / (1.0 + e2w)
    w1_ref[...] = w1
    w2_ref[...] = 1.0 - w1
    base_s[...] = base_s[...] + jnp.sum(both, axis=0, keepdims=True)

    @pl.when(c == (L // RC) - 1)
    def _():
        cnt = base_s[...].astype(jnp.int32)
        nt = (cnt + (TILE - 1)) >> 8
        pc = (nt << 8).astype(_F32)
        i0 = lax.broadcasted_iota(jnp.int32, (NEXP, NEXP), 0)
        i1 = lax.broadcasted_iota(jnp.int32, (NEXP, NEXP), 1)
        lt = (i0 < i1).astype(_F32)
        ps = lax.dot_general(pc, lt, (((1,), (0,)), ((), ())),
                             preferred_element_type=_F32)
        psi = ps.astype(jnp.int32)
        ps16_ref[...] = jnp.concatenate(
            [psi, jnp.zeros((1, NEXP), jnp.int32)], axis=1)
        pst_ref[...] = psi >> 8
        nt_ref[...] = nt


def _router(h, wgate):
    cspec = lambda: pl.BlockSpec((RC, 1), lambda c: (c, 0))
    out = pl.pallas_call(
        _router_body,
        grid=(L // RC,),
        in_specs=[
            pl.BlockSpec((RC, DIM), lambda c: (c, 0)),
            pl.BlockSpec((NEXP, DIM), lambda c: (0, 0)),
        ],
        out_specs=[
            cspec(), cspec(), cspec(), cspec(), cspec(), cspec(),
            pl.BlockSpec((1, 2 * NEXP), lambda c: (0, 0)),
            pl.BlockSpec((1, NEXP), lambda c: (0, 0)),
            pl.BlockSpec((1, NEXP), lambda c: (0, 0)),
        ],
        out_shape=[
            jax.ShapeDtypeStruct((L, 1), jnp.int32),
            jax.ShapeDtypeStruct((L, 1), jnp.int32),
            jax.ShapeDtypeStruct((L, 1), jnp.int32),
            jax.ShapeDtypeStruct((L, 1), jnp.int32),
            jax.ShapeDtypeStruct((L, 1), _F32),
            jax.ShapeDtypeStruct((L, 1), _F32),
            jax.ShapeDtypeStruct((1, 2 * NEXP), jnp.int32),
            jax.ShapeDtypeStruct((1, NEXP), jnp.int32),
            jax.ShapeDtypeStruct((1, NEXP), jnp.int32),
        ],
        scratch_shapes=[
            pltpu.VMEM((1, NEXP), _F32),
        ],
    )(h, wgate)
    e1, e2, rk1, rk2, w1, w2, ps16, pst, nt = out
    return (e1.reshape(L), e2.reshape(L), rk1.reshape(L), rk2.reshape(L),
            w1.reshape(L), w2.reshape(L), ps16.reshape(2 * NEXP),
            pst.reshape(NEXP), nt.reshape(NEXP))


def _sc_mesh():
    from jax.experimental.pallas import tpu_sc as plsc
    return plsc.VectorSubcoreMesh(core_axis_name="c", subcore_axis_name="s")


def _sc_wid():
    return lax.axis_index("s") * 2 + lax.axis_index("c")


def _pos_body(e1_ref, e2_ref, rk1_ref, rk2_ref, ps16_ref, pp1_ref, pp2_ref):
    ps = ps16_ref[...][:, :NEXP]
    def fin(e_ref, rk_ref, out_ref):
        oh = (lax.broadcasted_iota(jnp.int32, (RC, NEXP), 1)
              == e_ref[...]).astype(jnp.int32)
        out_ref[...] = rk_ref[...] + jnp.sum(oh * ps, axis=1, keepdims=True)
    fin(e1_ref, rk1_ref, pp1_ref)
    fin(e2_ref, rk2_ref, pp2_ref)


def _pos_finalize(e1, e2, rk1, rk2, ps16):
    cspec = lambda: pl.BlockSpec((RC, 1), lambda c: (c, 0))
    pp1, pp2 = pl.pallas_call(
        _pos_body,
        grid=(L // RC,),
        in_specs=[cspec(), cspec(), cspec(), cspec(),
                  pl.BlockSpec((1, 2 * NEXP), lambda c: (0, 0))],
        out_specs=[cspec(), cspec()],
        out_shape=[jax.ShapeDtypeStruct((L, 1), jnp.int32),
                   jax.ShapeDtypeStruct((L, 1), jnp.int32)],
    )(e1.reshape(L, 1), e2.reshape(L, 1), rk1.reshape(L, 1),
      rk2.reshape(L, 1), ps16.reshape(1, 2 * NEXP))
    return pp1.reshape(L), pp2.reshape(L)


def _dispatch(pp1, pp2, w1, w2):
    tpw = L // NW

    @functools.partial(
        pl.kernel,
        mesh=_sc_mesh(),
        out_type=[
            jax.ShapeDtypeStruct((TOTAL,), jnp.int32),
            jax.ShapeDtypeStruct((TOTAL,), _F32),
        ],
        scratch_types=[
            pltpu.VMEM((tpw,), jnp.int32),
            pltpu.VMEM((tpw,), jnp.int32),
            pltpu.VMEM((tpw,), _F32),
            pltpu.VMEM((tpw,), _F32),
            pltpu.VMEM((tpw,), jnp.int32),
            pltpu.SemaphoreType.DMA,
        ],
    )
    def k(pp1_hbm, pp2_hbm, w1_hbm, w2_hbm, stok_hbm, sw_hbm,
          p1_v, p2_v, w1_v, w2_v, tok_v, sem):
        base = _sc_wid() * tpw
        pltpu.sync_copy(pp1_hbm.at[pl.ds(base, tpw)], p1_v)
        pltpu.sync_copy(pp2_hbm.at[pl.ds(base, tpw)], p2_v)
        pltpu.sync_copy(w1_hbm.at[pl.ds(base, tpw)], w1_v)
        pltpu.sync_copy(w2_hbm.at[pl.ds(base, tpw)], w2_v)
        for j in range(tpw // 16):
            tok_v[pl.ds(j * 16, 16)] = (base + j * 16
                                        + lax.iota(jnp.int32, 16))
        pltpu.async_copy(tok_v, stok_hbm.at[p1_v], sem).wait()
        pltpu.async_copy(tok_v, stok_hbm.at[p2_v], sem).wait()
        pltpu.async_copy(w1_v, sw_hbm.at[p1_v], sem).wait()
        pltpu.async_copy(w2_v, sw_hbm.at[p2_v], sem).wait()

    return k(pp1, pp2, w1, w2)


def _gather_rows(stok, h):
    rpw = NDATA // NW
    sub = 64

    @functools.partial(
        pl.kernel,
        mesh=_sc_mesh(),
        out_type=jax.ShapeDtypeStruct((NDATA, DIM), _F32),
        scratch_types=[
            pltpu.VMEM((sub,), jnp.int32),
            pltpu.VMEM((sub, DIM), _F32),
            pltpu.SemaphoreType.DMA,
        ],
    )
    def k(stok_hbm, h_hbm, xg_hbm, idx_v, row_v, sem):
        base = _sc_wid() * rpw
        for s in range(rpw // sub):
            b = base + s * sub
            pltpu.sync_copy(stok_hbm.at[pl.ds(b, sub)], idx_v)
            for j in range(sub // 16):
                v = idx_v[pl.ds(j * 16, 16)]
                idx_v[pl.ds(j * 16, 16)] = jnp.minimum(
                    jnp.maximum(v, 0), L - 1)
            pltpu.async_copy(h_hbm.at[idx_v], row_v, sem).wait()
            pltpu.sync_copy(row_v, xg_hbm.at[pl.ds(b, sub)])

    return k(stok, h)


def _ffn_body(e, pst_ref, nt_ref, xg_ref, wg_ref, wu_ref, wd_ref, sw_ref,
              ywin_ref, yw_ref):
    t = pl.program_id(0)

    @pl.when(t < nt_ref[e])
    def _():
        xt = xg_ref[...]
        gate = _silu(_ntdot(xt, wg_ref[...]))
        up = _ntdot(xt, wu_ref[...])
        ffn = _ntdot(gate * up, wd_ref[...])
        r0 = lax.broadcasted_iota(jnp.int32, (TILE, TILE), 0)
        r1 = lax.broadcasted_iota(jnp.int32, (TILE, TILE), 1)
        eye = (r0 == r1).astype(_F32)
        w_col = _ntdot(eye, sw_ref[...][0])
        yw_ref[...] = ffn * w_col


def _ffn_tile_idx(e, t, pst, nt):
    return jnp.clip(pst[e] + jnp.minimum(t, nt[e] - 1), 0, DUMMY - 1)


def _moe_ffn(xg, sw, pst, nt, experts):
    sw2 = sw.reshape(TOTAL // TILE, 1, TILE)
    yw = None
    for e in range(NEXP):
        ep = experts[e]
        in_specs = [
            pl.BlockSpec((TILE, DIM),
                         functools.partial(lambda e_, t, pst_, nt_:
                                           (_ffn_tile_idx(e_, t, pst_, nt_), 0), e)),
            pl.BlockSpec((FFI, DIM), lambda t, pst_, nt_: (0, 0)),
            pl.BlockSpec((FFI, DIM), lambda t, pst_, nt_: (0, 0)),
            pl.BlockSpec((DIM, FFI), lambda t, pst_, nt_: (0, 0)),
            pl.BlockSpec((1, 1, TILE),
                         functools.partial(lambda e_, t, pst_, nt_:
                                           (_ffn_tile_idx(e_, t, pst_, nt_), 0, 0), e)),
            pl.BlockSpec((TILE, DIM), lambda t, pst_, nt_: (DUMMY, 0)),
        ]
        out_spec = pl.BlockSpec(
            (TILE, DIM),
            functools.partial(lambda e_, t, pst_, nt_:
                              (jnp.where(t < nt_[e_], pst_[e_] + t, DUMMY), 0), e))
        if yw is None:
            yw = jnp.zeros((TOTAL, DIM), _F32)
        yw = pl.pallas_call(
            functools.partial(_ffn_body, e),
            grid_spec=pltpu.PrefetchScalarGridSpec(
                num_scalar_prefetch=2,
                grid=(NT_MAX,),
                in_specs=in_specs,
                out_specs=out_spec,
            ),
            out_shape=jax.ShapeDtypeStruct((TOTAL, DIM), _F32),
            input_output_aliases={7: 0},
        )(pst, nt, xg, ep['Wg'], ep['Wu'], ep['Wd'], sw2, yw)
    return yw


def _combine(pp1, pp2, h, yw):
    tpw = L // NW
    sub = 16

    @functools.partial(
        pl.kernel,
        mesh=_sc_mesh(),
        out_type=jax.ShapeDtypeStruct((L, DIM), _F32),
        scratch_types=[
            pltpu.VMEM((tpw,), jnp.int32),
            pltpu.VMEM((tpw,), jnp.int32),
            pltpu.VMEM((sub, DIM), _F32),
            pltpu.VMEM((sub, DIM), _F32),
            pltpu.VMEM((sub, DIM), _F32),
            pltpu.SemaphoreType.DMA,
        ],
    )
    def k(pp1_hbm, pp2_hbm, h_hbm, yw_hbm, out_hbm,
          p1_v, p2_v, acc_v, g1_v, g2_v, sem):
        base = _sc_wid() * tpw
        pltpu.sync_copy(pp1_hbm.at[pl.ds(base, tpw)], p1_v)
        pltpu.sync_copy(pp2_hbm.at[pl.ds(base, tpw)], p2_v)
        for s in range(tpw // sub):
            b = base + s * sub
            i1 = p1_v[pl.ds(s * sub, sub)]
            i2 = p2_v[pl.ds(s * sub, sub)]
            cp1 = pltpu.async_copy(yw_hbm.at[i1], g1_v, sem)
            cp1.wait()
            cp2 = pltpu.async_copy(yw_hbm.at[i2], g2_v, sem)
            cp2.wait()
            pltpu.sync_copy(h_hbm.at[pl.ds(b, sub)], acc_v)
            for r in range(sub):
                def body(ci, _):
                    off = ci * 16
                    acc_v[r, pl.ds(off, 16)] = (
                        acc_v[r, pl.ds(off, 16)]
                        + g1_v[r, pl.ds(off, 16)]
                        + g2_v[r, pl.ds(off, 16)])
                    return 0
                lax.fori_loop(0, DIM // 16, body, 0)
            pltpu.sync_copy(acc_v, out_hbm.at[pl.ds(b, sub)])

    return k(pp1, pp2, h, yw)


def _moe_block(h, mp):
    e1, e2, rk1, rk2, w1, w2, ps16, pst, nt = _router(h, mp['W_gate'])
    pp1, pp2 = _pos_finalize(e1, e2, rk1, rk2, ps16)
    stok, sw = _dispatch(pp1, pp2, w1, w2)
    xg = _gather_rows(stok, h)
    yw = _moe_ffn(xg, sw, pst, nt, mp['experts'])
    return _combine(pp1, pp2, h, yw)


def _head_body(h_ref, w_ref, out_ref):
    out_ref[...] = jax.nn.sigmoid(_ntdot(h_ref[...], w_ref[...]))


def _head(h, w_head):
    return pl.pallas_call(
        _head_body,
        grid=(L // RC,),
        in_specs=[
            pl.BlockSpec((RC, DIM), lambda r: (r, 0)),
            pl.BlockSpec((DIM, DIM), lambda r: (0, 0)),
        ],
        out_specs=pl.BlockSpec((RC, DIM), lambda r: (r, 0)),
        out_shape=jax.ShapeDtypeStruct((L, DIM), _F32),
    )(h, w_head)


def kernel(x, params):
    h = x.reshape(L, DIM)
    for i in range(len(params['blocks'])):
        h = _mamba_block(h, params['blocks'][i])
        h = _moe_block(h, params['moes'][i])
    h = _head(h, params['W_head'])
    return h.reshape(x.shape)


# speedup vs baseline: 1.0102x; 1.0102x over previous
import functools

import jax
import jax.numpy as jnp
from jax import lax
from jax.experimental import pallas as pl
from jax.experimental.pallas import tpu as pltpu

L = 2048
DIM = 1024
DIN = 2048
DSTATE = 16
DTRANK = 64
DCONV = 4
NEXP = 8
FFI = 2048
LC = 64
RC = 256

_F32 = jnp.float32
_NT = (((1,), (1,)), ((), ()))


def _silu(v):
    return v * jax.nn.sigmoid(v)


def _softplus(v):
    return jnp.maximum(v, 0.0) + jnp.log1p(jnp.exp(-jnp.abs(v)))


def _ntdot(a, b):
    return lax.dot_general(a, b, _NT, preferred_element_type=_F32)


_BF16 = jnp.bfloat16


def _ntdot16(a, b16):
    return lax.dot_general(a.astype(_BF16), b16, _NT,
                           preferred_element_type=_F32)


def _mamba_body(x_ref, w_in_ref, conv_w_ref, conv_b_ref, wx_ref,
                w_dt_ref, b_dt_ref, alog_ref, dd_ref, w_out_ref,
                out_ref, tail_ref, state_ref, da_s, dbu_s, st_s):
    c = pl.program_id(0)

    @pl.when(c == 0)
    def _():
        tail_ref[...] = jnp.zeros_like(tail_ref)
        state_ref[...] = jnp.zeros_like(state_ref)

    xch = x_ref[...]
    xz = _ntdot16(xch, w_in_ref[...])
    xc = xz[:, :DIN]
    res = xz[:, DIN:]

    ext = jnp.concatenate([tail_ref[...], xc], axis=0)
    tail_ref[...] = xc[LC - (DCONV - 1):, :]
    conv = conv_b_ref[...]
    for k in range(DCONV):
        conv = conv + ext[k:k + LC, :] * conv_w_ref[k:k + 1, :]
    xcs = _silu(conv)

    x_dbl = _ntdot(xcs, wx_ref[...])
    delta_r = x_dbl[:, :DTRANK]
    bm = x_dbl[:, DTRANK:DTRANK + DSTATE]
    cm = x_dbl[:, DTRANK + DSTATE:]
    delta = _softplus(_ntdot(delta_r, w_dt_ref[...]) + b_dt_ref[...])
    u = delta * xcs

    a2 = -jnp.exp(alog_ref[...])
    da_s[...] = jnp.exp(delta[:, None, :] * a2[None, :, :])
    dbu_s[...] = bm[:, :, None] * u[:, None, :]

    def step(l, _):
        st = (da_s[pl.ds(l, 1)][0] * state_ref[...]
              + dbu_s[pl.ds(l, 1)][0])
        state_ref[...] = st
        st_s[pl.ds(l, 1)] = st[None]
        return 0

    lax.fori_loop(0, LC, step, 0, unroll=False)

    y = jnp.sum(st_s[...] * cm[:, :, None], axis=1)
    y = y + xcs * dd_ref[...]
    y = y * _silu(res)
    out_ref[...] = _ntdot16(y, w_out_ref[...]) + xch


def _mamba_block(h, bp):
    conv_w_t = bp['conv_w'].T
    conv_b = bp['conv_b'].reshape(1, DIN)
    b_dt = bp['b_dt'].reshape(1, DIN)
    alog_t = bp['A_log'].T
    dd = bp['D'].reshape(1, DIN)

    grid = L // LC
    full = lambda shape: pl.BlockSpec(shape, lambda c: (0,) * len(shape))
    return pl.pallas_call(
        _mamba_body,
        grid=(grid,),
        in_specs=[
            pl.BlockSpec((LC, DIM), lambda c: (c, 0)),
            full((2 * DIN, DIM)),
            full((DCONV, DIN)),
            full((1, DIN)),
            full((DTRANK + 2 * DSTATE, DIN)),
            full((DIN, DTRANK)),
            full((1, DIN)),
            full((DSTATE, DIN)),
            full((1, DIN)),
            full((DIM, DIN)),
        ],
        out_specs=pl.BlockSpec((LC, DIM), lambda c: (c, 0)),
        out_shape=jax.ShapeDtypeStruct((L, DIM), _F32),
        scratch_shapes=[
            pltpu.VMEM((DCONV - 1, DIN), _F32),
            pltpu.VMEM((DSTATE, DIN), _F32),
            pltpu.VMEM((LC, DSTATE, DIN), _F32),
            pltpu.VMEM((LC, DSTATE, DIN), _F32),
            pltpu.VMEM((LC, DSTATE, DIN), _F32),
        ],
    )(h, bp['W_in'].astype(_BF16), conv_w_t, conv_b, bp['W_x'],
      bp['W_dt'], b_dt, alog_t, dd, bp['W_out'].astype(_BF16))


def _top2_weight(h, wgate, e):
    scores = _ntdot(h, wgate)
    ii = lax.broadcasted_iota(jnp.int32, scores.shape, 1)
    m1 = jnp.max(scores, axis=-1, keepdims=True)
    a1 = jnp.min(jnp.where(scores == m1, ii, NEXP), axis=-1, keepdims=True)
    s2 = jnp.where(ii == a1, -jnp.inf, scores)
    m2 = jnp.max(s2, axis=-1, keepdims=True)
    a2 = jnp.min(jnp.where(s2 == m2, ii, NEXP), axis=-1, keepdims=True)
    e2 = jnp.exp(m2 - m1)
    w1 = 1.0 / (1.0 + e2)
    w2 = 1.0 - w1
    return jnp.where(a1 == e, w1, 0.0) + jnp.where(a2 == e, w2, 0.0)


def _moe_exp_body(e, h_ref, acc_ref, wgate_ref, wg_ref, wu_ref, wd_ref,
                  out_ref):
    h = h_ref[...]
    we = _top2_weight(h, wgate_ref[...], e)
    gate = _silu(_ntdot16(h, wg_ref[...]))
    up = _ntdot16(h, wu_ref[...])
    ffn = _ntdot16(gate * up, wd_ref[...])
    out_ref[...] = acc_ref[...] + we * ffn


def _moe_block(h, mp):
    acc = h
    for e in range(NEXP):
        ep = mp['experts'][e]
        acc = pl.pallas_call(
            functools.partial(_moe_exp_body, e),
            grid=(L // RC,),
            in_specs=[
                pl.BlockSpec((RC, DIM), lambda r: (r, 0)),
                pl.BlockSpec((RC, DIM), lambda r: (r, 0)),
                pl.BlockSpec((NEXP, DIM), lambda r: (0, 0)),
                pl.BlockSpec((FFI, DIM), lambda r: (0, 0)),
                pl.BlockSpec((FFI, DIM), lambda r: (0, 0)),
                pl.BlockSpec((DIM, FFI), lambda r: (0, 0)),
            ],
            out_specs=pl.BlockSpec((RC, DIM), lambda r: (r, 0)),
            out_shape=jax.ShapeDtypeStruct((L, DIM), _F32),
        )(h, acc, mp['W_gate'], ep['Wg'].astype(_BF16),
          ep['Wu'].astype(_BF16), ep['Wd'].astype(_BF16))
    return acc


def _head_body(h_ref, w_ref, out_ref):
    out_ref[...] = jax.nn.sigmoid(_ntdot16(h_ref[...], w_ref[...]))


def _head(h, w_head):
    return pl.pallas_call(
        _head_body,
        grid=(L // RC,),
        in_specs=[
            pl.BlockSpec((RC, DIM), lambda r: (r, 0)),
            pl.BlockSpec((DIM, DIM), lambda r: (0, 0)),
        ],
        out_specs=pl.BlockSpec((RC, DIM), lambda r: (r, 0)),
        out_shape=jax.ShapeDtypeStruct((L, DIM), _F32),
    )(h, w_head.astype(_BF16))


def kernel(x, params):
    h = x.reshape(L, DIM)
    for i in range(len(params['blocks'])):
        h = _mamba_block(h, params['blocks'][i])
        h = _moe_block(h, params['moes'][i])
    h = _head(h, params['W_head'])
    return h.reshape(x.shape)
